# baseline (speedup 1.0000x reference)
import functools
import jax
import jax.numpy as jnp
from jax import lax
from jax.experimental import pallas as pl
from jax.experimental.pallas import tpu as pltpu
from jax.experimental.pallas import tpu_sc as plsc

N = 10000
E = 320000
C = 128
H = 256
L = 128
EB = 1280
NB = 1000

GW = 32
EPW = E // GW
CH = 200
NCH = EPW // CH


def _gather_body(x_hbm, row_hbm, col_hbm, xg_hbm, ea_hbm,
                 idxr_v, idxc_v, rows_v, crows_v, ea_v, sem):
    wid = lax.axis_index("s") * 2 + lax.axis_index("c")
    base = wid * EPW

    def chunk(k, carry):
        off = base + k * CH
        pltpu.sync_copy(row_hbm.at[pl.ds(off, CH)], idxr_v)
        pltpu.sync_copy(col_hbm.at[pl.ds(off, CH)], idxc_v)
        pltpu.async_copy(x_hbm.at[idxr_v], rows_v, sem).wait()
        pltpu.async_copy(x_hbm.at[idxc_v], crows_v, sem).wait()

        def ea_row(r, c2):
            d = rows_v[r, pl.ds(0, 16)] - crows_v[r, pl.ds(0, 16)]
            d = jnp.where(d > 0.5, d - 1.0, d)
            d = jnp.where(-d > 0.5, d + 1.0, d)
            ea_v[r, pl.ds(0, 16)] = d
            return c2

        lax.fori_loop(0, CH, ea_row, 0)
        pltpu.sync_copy(rows_v, xg_hbm.at[pl.ds(off, CH)])
        pltpu.sync_copy(ea_v, ea_hbm.at[pl.ds(off, CH)])
        return carry

    lax.fori_loop(0, NCH, chunk, 0)


def _edge_mlp_kernel(xg_ref, ea_ref,
                     w1a_ref, w1b_ref, b1_ref, w2_ref, b2_ref, w3_ref, b3_ref,
                     out_ref):
    feat = xg_ref[...]
    ea = ea_ref[...]
    h = jnp.dot(feat, w1a_ref[...], preferred_element_type=jnp.float32)
    h = h + jnp.dot(ea, w1b_ref[...], preferred_element_type=jnp.float32)
    h = jax.nn.relu(h + b1_ref[...])
    h = jax.nn.relu(jnp.dot(h, w2_ref[...], preferred_element_type=jnp.float32)
                    + b2_ref[...])
    out_ref[...] = (jnp.dot(h, w3_ref[...], preferred_element_type=jnp.float32)
                    + b3_ref[...])


def _segment_kernel(col_ref, val_ref,
                    s0, s1, s2, s3, s4, m0, m1, m2, m3, m4, cpk,
                    c0, c1, c2, c3, c4):
    s_refs = (s0, s1, s2, s3, s4)
    m_refs = (m0, m1, m2, m3, m4)
    c_refs = (c0, c1, c2, c3, c4)

    @pl.when(pl.program_id(0) == 0)
    def _():
        for r in s_refs + c_refs:
            r[...] = jnp.zeros_like(r)
        for r in m_refs:
            r[...] = jnp.full_like(r, -3.0e38)

    lanes = lax.broadcasted_iota(jnp.int32, (1, L), 1)

    def body(i, _):
        for j in range(5):
            e = i * 5 + j
            c = col_ref[0, e]
            v = val_ref[pl.ds(e, 1), :]
            s_refs[j][pl.ds(c, 1), :] += v
            m_refs[j][pl.ds(c, 1), :] = jnp.maximum(m_refs[j][pl.ds(c, 1), :], v)
            onehot = jnp.where(lanes == c % 8, 1.0, 0.0)
            c_refs[j][pl.ds(c // 8, 1), :] += onehot
        return 0

    lax.fori_loop(0, EB // 5, body, 0)

    @pl.when(pl.program_id(0) == pl.num_programs(0) - 1)
    def _():
        cpk[...] = ((c0[...] + c1[...]) + (c2[...] + c3[...])) + c4[...]


def _node_mlp_kernel(x_ref, s0, s1, s2, s3, s4, m0, m1, m2, m3, m4, cpk_ref,
                     wa_ref, wb_ref, wc_ref, wd_ref, b1_ref,
                     w2_ref, b2_ref, w3_ref, b3_ref, out_ref, cblk):
    for k in range(N // NB):
        @pl.when(pl.program_id(0) == k)
        def _():
            cblk[...] = cpk_ref[k * (NB // 8):(k + 1) * (NB // 8), :]

    r8 = lax.broadcasted_iota(jnp.int32, (NB, NB // 8), 0) // 8
    csel = lax.broadcasted_iota(jnp.int32, (NB, NB // 8), 1)
    rep = jnp.dot(jnp.where(csel == r8, 1.0, 0.0), cblk[...],
                  preferred_element_type=jnp.float32)
    lsel = lax.broadcasted_iota(jnp.int32, (NB, L), 1)
    n8 = lax.broadcasted_iota(jnp.int32, (NB, L), 0) % 8
    cnt = jnp.sum(jnp.where(lsel == n8, rep, 0.0), axis=1, keepdims=True)
    s = ((s0[...] + s1[...]) + (s2[...] + s3[...])) + s4[...]
    mraw = jnp.maximum(
        jnp.maximum(jnp.maximum(m0[...], m1[...]),
                    jnp.maximum(m2[...], m3[...])), m4[...])
    mean = s / jnp.maximum(cnt, 1.0)
    mx = jnp.where(cnt > 0, mraw, 0.0)
    h = jnp.dot(x_ref[...], wa_ref[...], preferred_element_type=jnp.float32)
    h = h + jnp.dot(mean, wb_ref[...], preferred_element_type=jnp.float32)
    h = h + jnp.dot(mx, wc_ref[...], preferred_element_type=jnp.float32)
    h = h + jnp.dot(s, wd_ref[...], preferred_element_type=jnp.float32)
    h = jax.nn.relu(h + b1_ref[...])
    h = jax.nn.relu(jnp.dot(h, w2_ref[...], preferred_element_type=jnp.float32)
                    + b2_ref[...])
    out_ref[...] = (jnp.dot(h, w3_ref[...], preferred_element_type=jnp.float32)
                    + b3_ref[...])


def kernel(x, edge_index, edge_attr, u, batch,
           m1_w1, m1_b1, m1_w2, m1_b2, m1_w3, m1_b3,
           m2_w1, m2_b1, m2_w2, m2_b2, m2_w3, m2_b3):
    row = edge_index[0]
    col = edge_index[1]
    w1a = m1_w1[:C]
    w1b = jnp.pad(m1_w1[C:], ((0, 13), (0, 0)))

    gather = functools.partial(
        pl.kernel,
        out_type=[jax.ShapeDtypeStruct((E, C), jnp.float32),
                  jax.ShapeDtypeStruct((E, 16), jnp.float32)],
        mesh=plsc.VectorSubcoreMesh(core_axis_name="c", subcore_axis_name="s"),
        scratch_types=[pltpu.VMEM((CH,), jnp.int32),
                       pltpu.VMEM((CH,), jnp.int32),
                       pltpu.VMEM((CH, C), jnp.float32),
                       pltpu.VMEM((CH, C), jnp.float32),
                       pltpu.VMEM((CH, 16), jnp.float32),
                       pltpu.SemaphoreType.DMA],
    )(_gather_body)
    xg, ea = gather(x, row, col)

    full = lambda shape: pl.BlockSpec(shape, lambda i: (0,) * len(shape))
    grid_e = E // EB

    out_e = pl.pallas_call(
        _edge_mlp_kernel,
        grid=(grid_e,),
        in_specs=[
            pl.BlockSpec((EB, C), lambda i: (i, 0)),
            pl.BlockSpec((EB, 16), lambda i: (i, 0)),
            full((C, H)), full((16, H)), full((1, H)),
            full((H, H)), full((1, H)), full((H, L)), full((1, L)),
        ],
        out_specs=pl.BlockSpec((EB, L), lambda i: (i, 0)),
        out_shape=jax.ShapeDtypeStruct((E, L), jnp.float32),
    )(xg, ea, w1a, w1b, m1_b1.reshape(1, H),
      m1_w2, m1_b2.reshape(1, H), m1_w3, m1_b3.reshape(1, L))

    segs = pl.pallas_call(
        _segment_kernel,
        grid=(grid_e,),
        in_specs=[
            pl.BlockSpec((1, EB), lambda i: (0, i), memory_space=pltpu.SMEM),
            pl.BlockSpec((EB, L), lambda i: (i, 0)),
        ],
        out_specs=[full((N, L))] * 10 + [full((N // 8, L))],
        out_shape=[jax.ShapeDtypeStruct((N, L), jnp.float32)] * 10
        + [jax.ShapeDtypeStruct((N // 8, L), jnp.float32)],
        scratch_shapes=[pltpu.VMEM((N // 8, L), jnp.float32)] * 5,
    )(col.reshape(1, E), out_e)

    grid_n = N // NB
    out_n = pl.pallas_call(
        _node_mlp_kernel,
        grid=(grid_n,),
        in_specs=[
            pl.BlockSpec((NB, C), lambda i: (i, 0)),
        ] + [pl.BlockSpec((NB, L), lambda i: (i, 0))] * 10 + [
            full((N // 8, L)),
            full((C, H)), full((L, H)), full((L, H)), full((L, H)), full((1, H)),
            full((H, H)), full((1, H)), full((H, L)), full((1, L)),
        ],
        out_specs=pl.BlockSpec((NB, L), lambda i: (i, 0)),
        out_shape=jax.ShapeDtypeStruct((N, L), jnp.float32),
        scratch_shapes=[pltpu.VMEM((NB // 8, L), jnp.float32)],
    )(x, *segs,
      m2_w1[:C], m2_w1[C:C + L], m2_w1[C + L:C + 2 * L], m2_w1[C + 2 * L:],
      m2_b1.reshape(1, H), m2_w2, m2_b2.reshape(1, H),
      m2_w3, m2_b3.reshape(1, L))

    return jnp.concatenate([x[:, :3], out_n], axis=1)


# speedup vs baseline: 1.6081x; 1.6081x over previous
import functools
import jax
import jax.numpy as jnp
from jax import lax
from jax.experimental import pallas as pl
from jax.experimental.pallas import tpu as pltpu
from jax.experimental.pallas import tpu_sc as plsc

N = 10000
E = 320000
C = 128
H = 256
L = 128
EB = 1280
NB = 1000

GW = 32
EPW = E // GW
CH = 200
NCH = EPW // CH


def _gather_body(x_hbm, row_hbm, col_hbm, xg_hbm, ea_hbm,
                 idxr_v, idxc_v, rows_v, crows_v, ea_v, sem):
    wid = lax.axis_index("s") * 2 + lax.axis_index("c")
    base = wid * EPW

    def chunk(k, carry):
        off = base + k * CH
        pltpu.sync_copy(row_hbm.at[pl.ds(off, CH)], idxr_v)
        pltpu.sync_copy(col_hbm.at[pl.ds(off, CH)], idxc_v)
        pltpu.async_copy(x_hbm.at[idxr_v], rows_v, sem).wait()
        pltpu.async_copy(x_hbm.at[idxc_v], crows_v, sem).wait()

        def ea_row(r, c2):
            d = rows_v[r, pl.ds(0, 16)] - crows_v[r, pl.ds(0, 16)]
            d = jnp.where(d > 0.5, d - 1.0, d)
            d = jnp.where(-d > 0.5, d + 1.0, d)
            ea_v[r, pl.ds(0, 16)] = d
            return c2

        lax.fori_loop(0, CH, ea_row, 0)
        pltpu.sync_copy(rows_v, xg_hbm.at[pl.ds(off, CH)])
        pltpu.sync_copy(ea_v, ea_hbm.at[pl.ds(off, CH)])
        return carry

    lax.fori_loop(0, NCH, chunk, 0)


def _edge_mlp_kernel(xg_ref, ea_ref,
                     w1a_ref, w1b_ref, b1_ref, w2_ref, b2_ref, w3_ref, b3_ref,
                     out_ref):
    feat = xg_ref[...]
    ea = ea_ref[...]
    h = jnp.dot(feat, w1a_ref[...], preferred_element_type=jnp.float32)
    h = h + jnp.dot(ea, w1b_ref[...], preferred_element_type=jnp.float32)
    h = jax.nn.relu(h + b1_ref[...])
    h = jax.nn.relu(jnp.dot(h, w2_ref[...], preferred_element_type=jnp.float32)
                    + b2_ref[...])
    out_ref[...] = (jnp.dot(h, w3_ref[...], preferred_element_type=jnp.float32)
                    + b3_ref[...])


def _segment_kernel(col_ref, val_ref,
                    s0, s1, s2, s3, s4, m0, m1, m2, m3, m4,
                    c0, c1, c2, c3):
    s_refs = (s0, s1, s2, s3, s4)
    m_refs = (m0, m1, m2, m3, m4)
    c_refs = (c0, c1, c2, c3)

    @pl.when(pl.program_id(0) == 0)
    def _():
        for r in s_refs:
            r[...] = jnp.zeros_like(r)
        for r in m_refs:
            r[...] = jnp.full_like(r, -3.0e38)
        def zinit(n, _):
            for r in c_refs:
                r[0, n] = 0
            return 0

        lax.fori_loop(0, N, zinit, 0)

    def body(i, _):
        for j in range(5):
            e = i * 5 + j
            c = col_ref[0, e]
            v = val_ref[pl.ds(e, 1), :]
            s_refs[j][pl.ds(c, 1), :] += v
            m_refs[j][pl.ds(c, 1), :] = jnp.maximum(m_refs[j][pl.ds(c, 1), :], v)
            c_refs[j % 4][0, c] += 1
        return 0

    lax.fori_loop(0, EB // 5, body, 0)


def _node_mlp_kernel(x_ref, s0, s1, s2, s3, s4, m0, m1, m2, m3, m4,
                     c0, c1, c2, c3,
                     wa_ref, wb_ref, wc_ref, wd_ref, b1_ref,
                     w2_ref, b2_ref, w3_ref, b3_ref, out_ref):
    cnt = ((c0[...] + c1[...]) + (c2[...] + c3[...])).astype(jnp.float32)
    s = ((s0[...] + s1[...]) + (s2[...] + s3[...])) + s4[...]
    mraw = jnp.maximum(
        jnp.maximum(jnp.maximum(m0[...], m1[...]),
                    jnp.maximum(m2[...], m3[...])), m4[...])
    mean = s / jnp.maximum(cnt, 1.0)
    mx = jnp.where(cnt > 0, mraw, 0.0)
    h = jnp.dot(x_ref[...], wa_ref[...], preferred_element_type=jnp.float32)
    h = h + jnp.dot(mean, wb_ref[...], preferred_element_type=jnp.float32)
    h = h + jnp.dot(mx, wc_ref[...], preferred_element_type=jnp.float32)
    h = h + jnp.dot(s, wd_ref[...], preferred_element_type=jnp.float32)
    h = jax.nn.relu(h + b1_ref[...])
    h = jax.nn.relu(jnp.dot(h, w2_ref[...], preferred_element_type=jnp.float32)
                    + b2_ref[...])
    out_ref[...] = (jnp.dot(h, w3_ref[...], preferred_element_type=jnp.float32)
                    + b3_ref[...])


def kernel(x, edge_index, edge_attr, u, batch,
           m1_w1, m1_b1, m1_w2, m1_b2, m1_w3, m1_b3,
           m2_w1, m2_b1, m2_w2, m2_b2, m2_w3, m2_b3):
    row = edge_index[0]
    col = edge_index[1]
    w1a = m1_w1[:C]
    w1b = jnp.pad(m1_w1[C:], ((0, 13), (0, 0)))

    gather = functools.partial(
        pl.kernel,
        out_type=[jax.ShapeDtypeStruct((E, C), jnp.float32),
                  jax.ShapeDtypeStruct((E, 16), jnp.float32)],
        mesh=plsc.VectorSubcoreMesh(core_axis_name="c", subcore_axis_name="s"),
        scratch_types=[pltpu.VMEM((CH,), jnp.int32),
                       pltpu.VMEM((CH,), jnp.int32),
                       pltpu.VMEM((CH, C), jnp.float32),
                       pltpu.VMEM((CH, C), jnp.float32),
                       pltpu.VMEM((CH, 16), jnp.float32),
                       pltpu.SemaphoreType.DMA],
    )(_gather_body)
    xg, ea = gather(x, row, col)

    full = lambda shape: pl.BlockSpec(shape, lambda i: (0,) * len(shape))
    grid_e = E // EB

    out_e = pl.pallas_call(
        _edge_mlp_kernel,
        grid=(grid_e,),
        in_specs=[
            pl.BlockSpec((EB, C), lambda i: (i, 0)),
            pl.BlockSpec((EB, 16), lambda i: (i, 0)),
            full((C, H)), full((16, H)), full((1, H)),
            full((H, H)), full((1, H)), full((H, L)), full((1, L)),
        ],
        out_specs=pl.BlockSpec((EB, L), lambda i: (i, 0)),
        out_shape=jax.ShapeDtypeStruct((E, L), jnp.float32),
    )(xg, ea, w1a, w1b, m1_b1.reshape(1, H),
      m1_w2, m1_b2.reshape(1, H), m1_w3, m1_b3.reshape(1, L))

    segs = pl.pallas_call(
        _segment_kernel,
        grid=(grid_e,),
        in_specs=[
            pl.BlockSpec((1, EB), lambda i: (0, i), memory_space=pltpu.SMEM),
            pl.BlockSpec((EB, L), lambda i: (i, 0)),
        ],
        out_specs=[full((N, L))] * 10
        + [pl.BlockSpec((1, N), lambda i: (0, 0),
                        memory_space=pltpu.SMEM)] * 4,
        out_shape=[jax.ShapeDtypeStruct((N, L), jnp.float32)] * 10
        + [jax.ShapeDtypeStruct((1, N), jnp.int32)] * 4,
    )(col.reshape(1, E), out_e)
    cnts = [c.reshape(N, 1) for c in segs[10:]]

    grid_n = N // NB
    out_n = pl.pallas_call(
        _node_mlp_kernel,
        grid=(grid_n,),
        in_specs=[
            pl.BlockSpec((NB, C), lambda i: (i, 0)),
        ] + [pl.BlockSpec((NB, L), lambda i: (i, 0))] * 10 + [
            pl.BlockSpec((NB, 1), lambda i: (i, 0))] * 4 + [
            full((C, H)), full((L, H)), full((L, H)), full((L, H)), full((1, H)),
            full((H, H)), full((1, H)), full((H, L)), full((1, L)),
        ],
        out_specs=pl.BlockSpec((NB, L), lambda i: (i, 0)),
        out_shape=jax.ShapeDtypeStruct((N, L), jnp.float32),
    )(x, *segs[:10], *cnts,
      m2_w1[:C], m2_w1[C:C + L], m2_w1[C + L:C + 2 * L], m2_w1[C + 2 * L:],
      m2_b1.reshape(1, H), m2_w2, m2_b2.reshape(1, H),
      m2_w3, m2_b3.reshape(1, L))

    return jnp.concatenate([x[:, :3], out_n], axis=1)


# speedup vs baseline: 1.6440x; 1.0223x over previous
import functools
import jax
import jax.numpy as jnp
from jax import lax
from jax.experimental import pallas as pl
from jax.experimental.pallas import tpu as pltpu
from jax.experimental.pallas import tpu_sc as plsc

N = 10000
E = 320000
C = 128
H = 256
L = 128
EB = 1280
NB = 1000

GW = 32
EPW = E // GW
CH = 200
NCH = EPW // CH


def _gather_body(x_hbm, row_hbm, col_hbm, xg_hbm, ea_hbm,
                 idxr_v, idxc_v, rows_v, crows_v, ea_v, sem):
    wid = lax.axis_index("s") * 2 + lax.axis_index("c")
    base = wid * EPW

    def chunk(k, carry):
        off = base + k * CH
        pltpu.sync_copy(row_hbm.at[pl.ds(off, CH)], idxr_v)
        pltpu.sync_copy(col_hbm.at[pl.ds(off, CH)], idxc_v)
        pltpu.async_copy(x_hbm.at[idxr_v], rows_v, sem).wait()
        pltpu.async_copy(x_hbm.at[idxc_v], crows_v, sem).wait()

        def ea_row(r, c2):
            d = rows_v[r, pl.ds(0, 16)] - crows_v[r, pl.ds(0, 16)]
            d = jnp.where(d > 0.5, d - 1.0, d)
            d = jnp.where(-d > 0.5, d + 1.0, d)
            ea_v[r, pl.ds(0, 16)] = d
            return c2

        lax.fori_loop(0, CH, ea_row, 0)
        pltpu.sync_copy(rows_v, xg_hbm.at[pl.ds(off, CH)])
        pltpu.sync_copy(ea_v, ea_hbm.at[pl.ds(off, CH)])
        return carry

    lax.fori_loop(0, NCH, chunk, 0)


def _edge_mlp_kernel(xg_ref, ea_ref,
                     w1a_ref, w1b_ref, b1_ref, w2_ref, b2_ref, w3_ref, b3_ref,
                     out_ref):
    feat = xg_ref[...]
    ea = ea_ref[...]
    h = jnp.dot(feat, w1a_ref[...], preferred_element_type=jnp.float32)
    h = h + jnp.dot(ea, w1b_ref[...], preferred_element_type=jnp.float32)
    h = jax.nn.relu(h + b1_ref[...])
    h = jax.nn.relu(jnp.dot(h, w2_ref[...], preferred_element_type=jnp.float32)
                    + b2_ref[...])
    out_ref[...] = (jnp.dot(h, w3_ref[...], preferred_element_type=jnp.float32)
                    + b3_ref[...])


def _segment_kernel(col_ref, val_ref,
                    s0, s1, s2, s3, m0, m1, m2, m3,
                    c0, c1):
    s_refs = (s0, s1, s2, s3)
    m_refs = (m0, m1, m2, m3)
    c_refs = (c0, c1)

    @pl.when(pl.program_id(0) == 0)
    def _():
        for r in s_refs:
            r[...] = jnp.zeros_like(r)
        for r in m_refs:
            r[...] = jnp.full_like(r, -3.0e38)
        def zinit(n, _):
            for r in c_refs:
                r[0, n] = 0
            return 0

        lax.fori_loop(0, N, zinit, 0)

    def body(i, _):
        for j in range(4):
            e = i * 4 + j
            c = col_ref[0, e]
            v = val_ref[pl.ds(e, 1), :]
            s_refs[j][pl.ds(c, 1), :] += v
            m_refs[j][pl.ds(c, 1), :] = jnp.maximum(m_refs[j][pl.ds(c, 1), :], v)
            c_refs[j % 2][0, c] += 1
        return 0

    lax.fori_loop(0, EB // 4, body, 0)


def _node_mlp_kernel(x_ref, s0, s1, s2, s3, m0, m1, m2, m3,
                     c0, c1,
                     wa_ref, wb_ref, wc_ref, wd_ref, b1_ref,
                     w2_ref, b2_ref, w3_ref, b3_ref, out_ref):
    cnt = (c0[...] + c1[...]).astype(jnp.float32)
    s = (s0[...] + s1[...]) + (s2[...] + s3[...])
    mraw = jnp.maximum(jnp.maximum(m0[...], m1[...]),
                       jnp.maximum(m2[...], m3[...]))
    mean = s / jnp.maximum(cnt, 1.0)
    mx = jnp.where(cnt > 0, mraw, 0.0)
    h = jnp.dot(x_ref[...], wa_ref[...], preferred_element_type=jnp.float32)
    h = h + jnp.dot(mean, wb_ref[...], preferred_element_type=jnp.float32)
    h = h + jnp.dot(mx, wc_ref[...], preferred_element_type=jnp.float32)
    h = h + jnp.dot(s, wd_ref[...], preferred_element_type=jnp.float32)
    h = jax.nn.relu(h + b1_ref[...])
    h = jax.nn.relu(jnp.dot(h, w2_ref[...], preferred_element_type=jnp.float32)
                    + b2_ref[...])
    out_ref[...] = (jnp.dot(h, w3_ref[...], preferred_element_type=jnp.float32)
                    + b3_ref[...])


def kernel(x, edge_index, edge_attr, u, batch,
           m1_w1, m1_b1, m1_w2, m1_b2, m1_w3, m1_b3,
           m2_w1, m2_b1, m2_w2, m2_b2, m2_w3, m2_b3):
    row = edge_index[0]
    col = edge_index[1]
    w1a = m1_w1[:C]
    w1b = jnp.pad(m1_w1[C:], ((0, 13), (0, 0)))

    gather = functools.partial(
        pl.kernel,
        out_type=[jax.ShapeDtypeStruct((E, C), jnp.float32),
                  jax.ShapeDtypeStruct((E, 16), jnp.float32)],
        mesh=plsc.VectorSubcoreMesh(core_axis_name="c", subcore_axis_name="s"),
        scratch_types=[pltpu.VMEM((CH,), jnp.int32),
                       pltpu.VMEM((CH,), jnp.int32),
                       pltpu.VMEM((CH, C), jnp.float32),
                       pltpu.VMEM((CH, C), jnp.float32),
                       pltpu.VMEM((CH, 16), jnp.float32),
                       pltpu.SemaphoreType.DMA],
    )(_gather_body)
    xg, ea = gather(x, row, col)

    full = lambda shape: pl.BlockSpec(shape, lambda i: (0,) * len(shape))
    grid_e = E // EB

    out_e = pl.pallas_call(
        _edge_mlp_kernel,
        grid=(grid_e,),
        in_specs=[
            pl.BlockSpec((EB, C), lambda i: (i, 0)),
            pl.BlockSpec((EB, 16), lambda i: (i, 0)),
            full((C, H)), full((16, H)), full((1, H)),
            full((H, H)), full((1, H)), full((H, L)), full((1, L)),
        ],
        out_specs=pl.BlockSpec((EB, L), lambda i: (i, 0)),
        out_shape=jax.ShapeDtypeStruct((E, L), jnp.float32),
    )(xg, ea, w1a, w1b, m1_b1.reshape(1, H),
      m1_w2, m1_b2.reshape(1, H), m1_w3, m1_b3.reshape(1, L))

    segs = pl.pallas_call(
        _segment_kernel,
        grid=(grid_e,),
        in_specs=[
            pl.BlockSpec((1, EB), lambda i: (0, i), memory_space=pltpu.SMEM),
            pl.BlockSpec((EB, L), lambda i: (i, 0)),
        ],
        out_specs=[full((N, L))] * 8
        + [pl.BlockSpec((1, N), lambda i: (0, 0),
                        memory_space=pltpu.SMEM)] * 2,
        out_shape=[jax.ShapeDtypeStruct((N, L), jnp.float32)] * 8
        + [jax.ShapeDtypeStruct((1, N), jnp.int32)] * 2,
    )(col.reshape(1, E), out_e)
    cnts = [c.reshape(N, 1) for c in segs[8:]]

    grid_n = N // NB
    out_n = pl.pallas_call(
        _node_mlp_kernel,
        grid=(grid_n,),
        in_specs=[
            pl.BlockSpec((NB, C), lambda i: (i, 0)),
        ] + [pl.BlockSpec((NB, L), lambda i: (i, 0))] * 8 + [
            pl.BlockSpec((NB, 1), lambda i: (i, 0))] * 2 + [
            full((C, H)), full((L, H)), full((L, H)), full((L, H)), full((1, H)),
            full((H, H)), full((1, H)), full((H, L)), full((1, L)),
        ],
        out_specs=pl.BlockSpec((NB, L), lambda i: (i, 0)),
        out_shape=jax.ShapeDtypeStruct((N, L), jnp.float32),
    )(x, *segs[:8], *cnts,
      m2_w1[:C], m2_w1[C:C + L], m2_w1[C + L:C + 2 * L], m2_w1[C + 2 * L:],
      m2_b1.reshape(1, H), m2_w2, m2_b2.reshape(1, H),
      m2_w3, m2_b3.reshape(1, L))

    return jnp.concatenate([x[:, :3], out_n], axis=1)
